# baseline (speedup 1.0000x reference)
import jax
import jax.numpy as jnp
from jax.experimental import pallas as pl
from jax.experimental.pallas import tpu as pltpu

_TN = 2048
_NS = 1
_PAD = 128


def _body(nn, *refs):
    x_refs = refs[:_NS]
    w1t_ref, b1_ref, wct_ref, bc_ref, out_ref, acc_ref = refs[_NS:]
    n = pl.program_id(1)
    psum = None
    for x_ref in x_refs:
        h1 = jnp.dot(x_ref[0], w1t_ref[...],
                     preferred_element_type=jnp.float32)
        h1 = jnp.maximum(h1 + b1_ref[...], 0.0)
        s = jnp.sum(h1, axis=0, keepdims=True)
        psum = s if psum is None else psum + s

    @pl.when(n == 0)
    def _():
        acc_ref[...] = psum

    @pl.when(n != 0)
    def _():
        acc_ref[...] += psum

    @pl.when(n == nn - 1)
    def _():
        row = jnp.dot(acc_ref[...], wct_ref[...],
                      preferred_element_type=jnp.float32) + bc_ref[...]
        out_ref[0] = row


def kernel(h, W1, b1, Wa, ba, Wb, bb, Wc, bc, Wcls, bcls):
    B, N, L0 = h.shape
    L1 = W1.shape[0]
    NC = Wcls.shape[0]
    nn = N // (_NS * _TN)

    w1t = W1.T
    b1r = b1.reshape(1, L1)
    wct = jnp.zeros((L1, _PAD), jnp.float32).at[:, :NC].set(Wcls.T)
    bcr = jnp.zeros((1, _PAD), jnp.float32).at[0, :NC].set(bcls)

    def _x_spec(s):
        return pl.BlockSpec((1, _TN, L0), lambda b, n, s=s: (b, n + s * nn, 0))

    out = pl.pallas_call(
        lambda *refs: _body(nn, *refs),
        grid=(B, nn),
        in_specs=[_x_spec(s) for s in range(_NS)] + [
            pl.BlockSpec((L0, L1), lambda b, n: (0, 0)),
            pl.BlockSpec((1, L1), lambda b, n: (0, 0)),
            pl.BlockSpec((L1, _PAD), lambda b, n: (0, 0)),
            pl.BlockSpec((1, _PAD), lambda b, n: (0, 0)),
        ],
        out_specs=pl.BlockSpec((1, 1, _PAD), lambda b, n: (b, 0, 0)),
        out_shape=jax.ShapeDtypeStruct((B, 1, _PAD), jnp.float32),
        scratch_shapes=[pltpu.VMEM((1, L1), jnp.float32)],
        compiler_params=pltpu.CompilerParams(
            dimension_semantics=("parallel", "arbitrary")),
    )(*([h] * _NS), w1t, b1r, wct, bcr)
    return out[:, 0, :NC]


# speedup vs baseline: 1.0022x; 1.0022x over previous
import jax
import jax.numpy as jnp
from jax.experimental import pallas as pl
from jax.experimental.pallas import tpu as pltpu

_TN = 2048
_PAD = 128


def _body(nn, N, x_ref, w1t_ref, nb1_ref, b1s_ref, wct_ref, bc_ref,
          out_ref, acc_ref):
    n = pl.program_id(1)
    h1 = jnp.dot(x_ref[0].astype(jnp.bfloat16), w1t_ref[...],
                 preferred_element_type=jnp.float32)
    r = jnp.maximum(h1, nb1_ref[...])
    part = jnp.sum(r.reshape(_TN // 8, 8, -1), axis=0)

    @pl.when(n == 0)
    def _():
        acc_ref[...] = part

    @pl.when(n != 0)
    def _():
        acc_ref[...] += part

    @pl.when(n == nn - 1)
    def _():
        m = jnp.sum(acc_ref[...], axis=0, keepdims=True) + b1s_ref[...]
        row = jnp.dot(m, wct_ref[...],
                      preferred_element_type=jnp.float32) + bc_ref[...]
        out_ref[0] = row


def kernel(h, W1, b1, Wa, ba, Wb, bb, Wc, bc, Wcls, bcls):
    B, N, L0 = h.shape
    L1 = W1.shape[0]
    NC = Wcls.shape[0]
    nn = N // _TN

    w1t = W1.T.astype(jnp.bfloat16)
    nb1 = (-b1).reshape(1, L1)
    b1s = (jnp.float32(N) * b1).reshape(1, L1)
    wct = jnp.zeros((L1, _PAD), jnp.float32).at[:, :NC].set(Wcls.T)
    bcr = jnp.zeros((1, _PAD), jnp.float32).at[0, :NC].set(bcls)

    out = pl.pallas_call(
        lambda *refs: _body(nn, N, *refs),
        grid=(B, nn),
        in_specs=[
            pl.BlockSpec((1, _TN, L0), lambda b, n: (b, n, 0)),
            pl.BlockSpec((L0, L1), lambda b, n: (0, 0)),
            pl.BlockSpec((1, L1), lambda b, n: (0, 0)),
            pl.BlockSpec((1, L1), lambda b, n: (0, 0)),
            pl.BlockSpec((L1, _PAD), lambda b, n: (0, 0)),
            pl.BlockSpec((1, _PAD), lambda b, n: (0, 0)),
        ],
        out_specs=pl.BlockSpec((1, 1, _PAD), lambda b, n: (b, 0, 0)),
        out_shape=jax.ShapeDtypeStruct((B, 1, _PAD), jnp.float32),
        scratch_shapes=[pltpu.VMEM((8, L1), jnp.float32)],
        compiler_params=pltpu.CompilerParams(
            dimension_semantics=("parallel", "arbitrary")),
    )(h, w1t, nb1, b1s, wct, bcr)
    return out[:, 0, :NC]


# speedup vs baseline: 1.0029x; 1.0007x over previous
import jax
import jax.numpy as jnp
from jax.experimental import pallas as pl
from jax.experimental.pallas import tpu as pltpu

_TN = 2048
_PAD = 128


def _body(nn, x_ref, w1t_ref, nb1_ref, b1s_ref, wct_ref, bc_ref,
          out_ref, acc_ref):
    i = pl.program_id(0)
    n = jax.lax.rem(i, nn)
    h1 = jnp.dot(x_ref[0].astype(jnp.bfloat16), w1t_ref[...],
                 preferred_element_type=jnp.float32)
    r = jnp.maximum(h1, nb1_ref[...])
    part = jnp.sum(r.reshape(_TN // 8, 8, -1), axis=0)

    @pl.when(n == 0)
    def _():
        acc_ref[...] = part

    @pl.when(n != 0)
    def _():
        acc_ref[...] += part

    @pl.when(n == nn - 1)
    def _():
        m = jnp.sum(acc_ref[...], axis=0, keepdims=True) + b1s_ref[...]
        row = jnp.dot(m, wct_ref[...],
                      preferred_element_type=jnp.float32) + bc_ref[...]
        out_ref[0] = row


def kernel(h, W1, b1, Wa, ba, Wb, bb, Wc, bc, Wcls, bcls):
    B, N, L0 = h.shape
    L1 = W1.shape[0]
    NC = Wcls.shape[0]
    nn = N // _TN

    w1t = W1.T.astype(jnp.bfloat16)
    nb1 = (-b1).reshape(1, L1)
    b1s = (jnp.float32(N) * b1).reshape(1, L1)
    wct = jnp.zeros((L1, _PAD), jnp.float32).at[:, :NC].set(Wcls.T)
    bcr = jnp.zeros((1, _PAD), jnp.float32).at[0, :NC].set(bcls)

    out = pl.pallas_call(
        lambda *refs: _body(nn, *refs),
        grid=(B * nn,),
        in_specs=[
            pl.BlockSpec((1, _TN, L0), lambda i: (i // nn, i % nn, 0)),
            pl.BlockSpec((L0, L1), lambda i: (0, 0)),
            pl.BlockSpec((1, L1), lambda i: (0, 0)),
            pl.BlockSpec((1, L1), lambda i: (0, 0)),
            pl.BlockSpec((L1, _PAD), lambda i: (0, 0)),
            pl.BlockSpec((1, _PAD), lambda i: (0, 0)),
        ],
        out_specs=pl.BlockSpec((1, 1, _PAD), lambda i: (i // nn, 0, 0)),
        out_shape=jax.ShapeDtypeStruct((B, 1, _PAD), jnp.float32),
        scratch_shapes=[pltpu.VMEM((8, L1), jnp.float32)],
        compiler_params=pltpu.CompilerParams(
            dimension_semantics=("arbitrary",)),
    )(h, w1t, nb1, b1s, wct, bcr)
    return out[:, 0, :NC]


# speedup vs baseline: 1.0778x; 1.0747x over previous
import jax
import jax.numpy as jnp
from jax.experimental import pallas as pl
from jax.experimental.pallas import tpu as pltpu

_TN = 2048

_TDN = (((1,), (1,)), ((), ()))


def _body(nn, N, x_ref, w1_ref, b1_ref, wcls_ref, bcls_ref,
          out_ref, acc_ref):
    i = pl.program_id(0)
    n = jax.lax.rem(i, nn)
    h1 = jax.lax.dot_general(x_ref[0].astype(jnp.bfloat16), w1_ref[...],
                             _TDN, preferred_element_type=jnp.float32)
    r = jnp.maximum(h1, -b1_ref[...])
    part = jnp.sum(r.reshape(_TN // 8, 8, -1), axis=0)

    @pl.when(n == 0)
    def _():
        acc_ref[...] = part

    @pl.when(n != 0)
    def _():
        acc_ref[...] += part

    @pl.when(n == nn - 1)
    def _():
        m = (jnp.sum(acc_ref[...], axis=0, keepdims=True)
             + jnp.float32(N) * b1_ref[...])
        row = jax.lax.dot_general(m, wcls_ref[...], _TDN,
                                  preferred_element_type=jnp.float32)
        out_ref[0] = row + bcls_ref[...]


def kernel(h, W1, b1, Wa, ba, Wb, bb, Wc, bc, Wcls, bcls):
    B, N, L0 = h.shape
    L1 = W1.shape[0]
    NC = Wcls.shape[0]
    nn = N // _TN

    w1b = W1.astype(jnp.bfloat16)

    out = pl.pallas_call(
        lambda *refs: _body(nn, N, *refs),
        grid=(B * nn,),
        in_specs=[
            pl.BlockSpec((1, _TN, L0), lambda i: (i // nn, i % nn, 0)),
            pl.BlockSpec((L1, L0), lambda i: (0, 0)),
            pl.BlockSpec((1, L1), lambda i: (0, 0)),
            pl.BlockSpec((NC, L1), lambda i: (0, 0)),
            pl.BlockSpec((1, NC), lambda i: (0, 0)),
        ],
        out_specs=pl.BlockSpec((1, 1, NC), lambda i: (i // nn, 0, 0)),
        out_shape=jax.ShapeDtypeStruct((B, 1, NC), jnp.float32),
        scratch_shapes=[pltpu.VMEM((8, L1), jnp.float32)],
        compiler_params=pltpu.CompilerParams(
            dimension_semantics=("arbitrary",)),
    )(h, w1b, b1.reshape(1, L1), Wcls, bcls.reshape(1, NC))
    return out[:, 0, :]


# speedup vs baseline: 1.1116x; 1.0314x over previous
import jax
import jax.numpy as jnp
from jax.experimental import pallas as pl
from jax.experimental.pallas import tpu as pltpu

_TN = 2048

_TDN = (((1,), (1,)), ((), ()))


def _body(nn, N, x_ref, w1_ref, b1_ref, wcls_ref, bcls_ref,
          out_ref, acc_ref, w1b_ref):
    i = pl.program_id(0)
    n = jax.lax.rem(i, nn)

    @pl.when(i == 0)
    def _():
        w1b_ref[...] = w1_ref[...].astype(jnp.bfloat16)

    h1 = jax.lax.dot_general(x_ref[0].astype(jnp.bfloat16), w1b_ref[...],
                             _TDN, preferred_element_type=jnp.float32)
    r = jnp.maximum(h1, -b1_ref[...])
    part = jnp.sum(r.reshape(_TN // 8, 8, -1), axis=0)

    @pl.when(n == 0)
    def _():
        acc_ref[...] = part

    @pl.when(n != 0)
    def _():
        acc_ref[...] += part

    @pl.when(n == nn - 1)
    def _():
        m = (jnp.sum(acc_ref[...], axis=0, keepdims=True)
             + jnp.float32(N) * b1_ref[...])
        row = jax.lax.dot_general(m, wcls_ref[...], _TDN,
                                  preferred_element_type=jnp.float32)
        out_ref[0] = row + bcls_ref[...]


def kernel(h, W1, b1, Wa, ba, Wb, bb, Wc, bc, Wcls, bcls):
    B, N, L0 = h.shape
    L1 = W1.shape[0]
    NC = Wcls.shape[0]
    nn = N // _TN

    out = pl.pallas_call(
        lambda *refs: _body(nn, N, *refs),
        grid=(B * nn,),
        in_specs=[
            pl.BlockSpec((1, _TN, L0), lambda i: (i // nn, i % nn, 0)),
            pl.BlockSpec((L1, L0), lambda i: (0, 0)),
            pl.BlockSpec((1, L1), lambda i: (0, 0)),
            pl.BlockSpec((NC, L1), lambda i: (0, 0)),
            pl.BlockSpec((1, NC), lambda i: (0, 0)),
        ],
        out_specs=pl.BlockSpec((1, 1, NC), lambda i: (i // nn, 0, 0)),
        out_shape=jax.ShapeDtypeStruct((B, 1, NC), jnp.float32),
        scratch_shapes=[pltpu.VMEM((8, L1), jnp.float32),
                        pltpu.VMEM((L1, L0), jnp.bfloat16)],
        compiler_params=pltpu.CompilerParams(
            dimension_semantics=("arbitrary",)),
    )(h, W1, b1.reshape(1, L1), Wcls, bcls.reshape(1, NC))
    return out[:, 0, :]


# speedup vs baseline: 1.1194x; 1.0070x over previous
import jax
import jax.numpy as jnp
from jax.experimental import pallas as pl
from jax.experimental.pallas import tpu as pltpu

_TN = 2048

_TDN = (((1,), (1,)), ((), ()))


def _body(nn, N, x_ref, w1_ref, b1_ref, wcls_ref, bcls_ref,
          out_ref, acc_ref, w1b_ref):
    i = pl.program_id(0)
    n = jax.lax.rem(i, nn)

    @pl.when(i == 0)
    def _():
        w1b_ref[...] = w1_ref[...].astype(jnp.bfloat16)

    part = None
    for rc in range(2):
        xc = x_ref[0, pl.ds(rc * (_TN // 2), _TN // 2), :].astype(jnp.bfloat16)
        h1 = jax.lax.dot_general(xc, w1b_ref[...], _TDN,
                                 preferred_element_type=jnp.float32)
        r = jnp.maximum(h1, -b1_ref[...])
        s = jnp.sum(r.reshape(_TN // 16, 8, -1), axis=0)
        part = s if part is None else part + s

    @pl.when(n == 0)
    def _():
        acc_ref[...] = part

    @pl.when(n != 0)
    def _():
        acc_ref[...] += part

    @pl.when(n == nn - 1)
    def _():
        m = (jnp.sum(acc_ref[...], axis=0, keepdims=True)
             + jnp.float32(N) * b1_ref[...])
        row = jax.lax.dot_general(m, wcls_ref[...], _TDN,
                                  preferred_element_type=jnp.float32)
        out_ref[0] = row + bcls_ref[...]


def kernel(h, W1, b1, Wa, ba, Wb, bb, Wc, bc, Wcls, bcls):
    B, N, L0 = h.shape
    L1 = W1.shape[0]
    NC = Wcls.shape[0]
    nn = N // _TN

    out = pl.pallas_call(
        lambda *refs: _body(nn, N, *refs),
        grid=(B * nn,),
        in_specs=[
            pl.BlockSpec((1, _TN, L0), lambda i: (i // nn, i % nn, 0)),
            pl.BlockSpec((L1, L0), lambda i: (0, 0)),
            pl.BlockSpec((1, L1), lambda i: (0, 0)),
            pl.BlockSpec((NC, L1), lambda i: (0, 0)),
            pl.BlockSpec((1, NC), lambda i: (0, 0)),
        ],
        out_specs=pl.BlockSpec((1, 1, NC), lambda i: (i // nn, 0, 0)),
        out_shape=jax.ShapeDtypeStruct((B, 1, NC), jnp.float32),
        scratch_shapes=[pltpu.VMEM((8, L1), jnp.float32),
                        pltpu.VMEM((L1, L0), jnp.bfloat16)],
        compiler_params=pltpu.CompilerParams(
            dimension_semantics=("arbitrary",)),
    )(h, W1, b1.reshape(1, L1), Wcls, bcls.reshape(1, NC))
    return out[:, 0, :]
